# baseline (device time: 126453 ns/iter reference)
import jax
import jax.numpy as jnp
from jax import lax
from jax.experimental import pallas as pl
from jax.experimental.pallas import tpu as pltpu

N_DEV = 4
HQ = 8
DH = 128
SQ = 1024
SKV = 1024
DMODEL = 1024
BLK = 64
SCALE = 0.08838834764831843


def kernel(x, Wq, K_ext, V_ext, Wo):
    my = lax.axis_index("i")

    xb = x[0].astype(jnp.bfloat16)
    Wqb = Wq.astype(jnp.bfloat16)
    Wob = Wo.astype(jnp.bfloat16)
    KT = jnp.transpose(K_ext[0], (1, 2, 0))
    KT = lax.dynamic_slice_in_dim(KT, my * HQ, HQ, 0).astype(jnp.bfloat16)
    Vh = jnp.transpose(V_ext[0], (1, 0, 2))
    Vh = lax.dynamic_slice_in_dim(Vh, my * HQ, HQ, 0).astype(jnp.bfloat16)

    def body(x_ref, wq_ref, kt_ref, v_ref, wo_ref, out_ref,
             bias_ref, comm_ref, send_sems, recv_sems):
        row = lax.broadcasted_iota(jnp.int32, (SQ, SKV), 0) // BLK
        col = lax.broadcasted_iota(jnp.int32, (SQ, SKV), 1) // BLK
        keep = (row == col) | (col == 0) | ((row + col) % 3 == 0)
        bias_ref[...] = jnp.where(keep, 0.0, -1e9)

        acc = jnp.zeros((SQ, DMODEL), jnp.float32)
        for h in range(HQ):
            q = jnp.dot(x_ref[...], wq_ref[:, h * DH:(h + 1) * DH],
                        preferred_element_type=jnp.float32).astype(jnp.bfloat16)
            s = jnp.dot(q, kt_ref[h], preferred_element_type=jnp.float32)
            s = s * SCALE + bias_ref[...]
            m = jnp.max(s, axis=1, keepdims=True)
            e = jnp.exp(s - m)
            den = jnp.sum(e, axis=1, keepdims=True)
            w = (e / den).astype(jnp.bfloat16)
            ctx = jnp.dot(w, v_ref[h],
                          preferred_element_type=jnp.float32).astype(jnp.bfloat16)
            acc = acc + jnp.dot(ctx, wo_ref[h * DH:(h + 1) * DH, :],
                                preferred_element_type=jnp.float32)
        out_ref[...] = acc
        comm_ref[0] = acc.astype(jnp.bfloat16)

        my_pos = lax.axis_index("i")
        left = lax.rem(my_pos + N_DEV - 1, N_DEV)
        right = lax.rem(my_pos + 1, N_DEV)

        barrier = pltpu.get_barrier_semaphore()
        for nbr in (left, right):
            pl.semaphore_signal(barrier, inc=1, device_id=(nbr,),
                                device_id_type=pl.DeviceIdType.MESH)
        pl.semaphore_wait(barrier, 2)

        for hop in range(N_DEV - 1):
            rdma = pltpu.make_async_remote_copy(
                src_ref=comm_ref.at[hop],
                dst_ref=comm_ref.at[hop + 1],
                send_sem=send_sems.at[hop],
                recv_sem=recv_sems.at[hop],
                device_id=(right,),
                device_id_type=pl.DeviceIdType.MESH,
            )
            rdma.start()
            rdma.wait()
            out_ref[...] += comm_ref[hop + 1].astype(jnp.float32)

    out = pl.pallas_call(
        body,
        out_shape=jax.ShapeDtypeStruct((SQ, DMODEL), jnp.float32),
        in_specs=[pl.BlockSpec(memory_space=pltpu.VMEM)] * 5,
        out_specs=pl.BlockSpec(memory_space=pltpu.VMEM),
        scratch_shapes=[
            pltpu.VMEM((SQ, SKV), jnp.float32),
            pltpu.VMEM((N_DEV, SQ, DMODEL), jnp.bfloat16),
            pltpu.SemaphoreType.DMA((N_DEV - 1,)),
            pltpu.SemaphoreType.DMA((N_DEV - 1,)),
        ],
        compiler_params=pltpu.CompilerParams(collective_id=0),
    )(xb, Wqb, KT, Vh, Wob)
    return out[None]


# device time: 61440 ns/iter; 2.0582x vs baseline; 2.0582x over previous
import jax
import jax.numpy as jnp
from jax import lax
from jax.experimental import pallas as pl
from jax.experimental.pallas import tpu as pltpu

N_DEV = 4
HQ = 8
DH = 128
SQ = 1024
SKV = 1024
DMODEL = 1024
R = SQ // N_DEV
BLK = 64
SCALE = 0.08838834764831843


def kernel(x, Wq, K_ext, V_ext, Wo):
    my = lax.axis_index("i")

    xb = x[0].astype(jnp.bfloat16)
    Wqb = Wq.astype(jnp.bfloat16)
    Wob = Wo.astype(jnp.bfloat16)
    KT = jnp.transpose(K_ext[0], (1, 2, 0))
    KT = lax.dynamic_slice_in_dim(KT, my * HQ, HQ, 0).astype(jnp.bfloat16)
    Vh = jnp.transpose(V_ext[0], (1, 0, 2))
    Vh = lax.dynamic_slice_in_dim(Vh, my * HQ, HQ, 0).astype(jnp.bfloat16)

    def body(x_ref, wq_ref, kt_ref, v_ref, wo_ref, out_ref, bias_ref,
             red_send, red_buf, bc_send, bc_buf,
             red_send_sems, red_recv_sems, bc_send_sems, bc_recv_sems):
        my_pos = lax.axis_index("i")

        barrier = pltpu.get_barrier_semaphore()
        for j in range(1, N_DEV):
            pl.semaphore_signal(
                barrier, inc=1,
                device_id=(lax.rem(my_pos + j, N_DEV),),
                device_id_type=pl.DeviceIdType.MESH,
            )

        row = lax.broadcasted_iota(jnp.int32, (SQ, SKV), 0) // BLK
        col = lax.broadcasted_iota(jnp.int32, (SQ, SKV), 1) // BLK
        keep = (row == col) | (col == 0) | ((row + col) % 3 == 0)
        bias_ref[...] = jnp.where(keep, 0.0, -1e9)

        def compute_chunk(c):
            row0 = c * R
            x_rows = x_ref[pl.ds(row0, R), :]
            bias = bias_ref[pl.ds(row0, R), :]
            q_all = jnp.dot(x_rows, wq_ref[...],
                            preferred_element_type=jnp.float32)
            q_all = (q_all * SCALE).astype(jnp.bfloat16)
            ctxs = []
            for h in range(HQ):
                s = jnp.dot(q_all[:, h * DH:(h + 1) * DH], kt_ref[h],
                            preferred_element_type=jnp.float32)
                s = s + bias
                m = jnp.max(s, axis=1, keepdims=True)
                e = jnp.exp(s - m)
                den = jnp.sum(e, axis=1, keepdims=True)
                w = (e / den).astype(jnp.bfloat16)
                ctxs.append(jnp.dot(w, v_ref[h],
                                    preferred_element_type=jnp.float32)
                            .astype(jnp.bfloat16))
            ctx_all = jnp.concatenate(ctxs, axis=1)
            return jnp.dot(ctx_all, wo_ref[...],
                           preferred_element_type=jnp.float32)

        red_rdmas = []
        for j in range(1, N_DEV):
            c = lax.rem(my_pos + j, N_DEV)
            part = compute_chunk(c)
            red_send[j - 1] = part.astype(jnp.bfloat16)
            if j == 1:
                pl.semaphore_wait(barrier, N_DEV - 1)
            rdma = pltpu.make_async_remote_copy(
                src_ref=red_send.at[j - 1],
                dst_ref=red_buf.at[j - 1],
                send_sem=red_send_sems.at[j - 1],
                recv_sem=red_recv_sems.at[j - 1],
                device_id=(c,),
                device_id_type=pl.DeviceIdType.MESH,
            )
            rdma.start()
            red_rdmas.append(rdma)

        own = compute_chunk(my_pos)
        total = own
        for k in range(N_DEV - 1):
            red_rdmas[k].wait_recv()
            total = total + red_buf[k].astype(jnp.float32)
        out_ref[pl.ds(my_pos * R, R), :] = total

        bc_send[...] = total.astype(jnp.bfloat16)
        bc_rdmas = []
        for j in range(1, N_DEV):
            rdma = pltpu.make_async_remote_copy(
                src_ref=bc_send,
                dst_ref=bc_buf.at[j - 1],
                send_sem=bc_send_sems.at[j - 1],
                recv_sem=bc_recv_sems.at[j - 1],
                device_id=(lax.rem(my_pos + j, N_DEV),),
                device_id_type=pl.DeviceIdType.MESH,
            )
            rdma.start()
            bc_rdmas.append(rdma)
        for k in range(N_DEV - 1):
            owner = lax.rem(my_pos + N_DEV - 1 - k, N_DEV)
            bc_rdmas[k].wait_recv()
            out_ref[pl.ds(owner * R, R), :] = bc_buf[k].astype(jnp.float32)

        for rdma in red_rdmas + bc_rdmas:
            rdma.wait_send()

    out = pl.pallas_call(
        body,
        out_shape=jax.ShapeDtypeStruct((SQ, DMODEL), jnp.float32),
        in_specs=[pl.BlockSpec(memory_space=pltpu.VMEM)] * 5,
        out_specs=pl.BlockSpec(memory_space=pltpu.VMEM),
        scratch_shapes=[
            pltpu.VMEM((SQ, SKV), jnp.float32),
            pltpu.VMEM((N_DEV - 1, R, DMODEL), jnp.bfloat16),
            pltpu.VMEM((N_DEV - 1, R, DMODEL), jnp.bfloat16),
            pltpu.VMEM((R, DMODEL), jnp.bfloat16),
            pltpu.VMEM((N_DEV - 1, R, DMODEL), jnp.bfloat16),
            pltpu.SemaphoreType.DMA((N_DEV - 1,)),
            pltpu.SemaphoreType.DMA((N_DEV - 1,)),
            pltpu.SemaphoreType.DMA((N_DEV - 1,)),
            pltpu.SemaphoreType.DMA((N_DEV - 1,)),
        ],
        compiler_params=pltpu.CompilerParams(collective_id=0),
    )(xb, Wqb, KT, Vh, Wob)
    return out[None]


# device time: 61377 ns/iter; 2.0603x vs baseline; 1.0010x over previous
import jax
import jax.numpy as jnp
from jax import lax
from jax.experimental import pallas as pl
from jax.experimental.pallas import tpu as pltpu

N_DEV = 4
HQ = 8
DH = 128
SQ = 1024
SKV = 1024
DMODEL = 1024
R = SQ // N_DEV
BLK = 64
SCALE = 0.08838834764831843


def kernel(x, Wq, K_ext, V_ext, Wo):
    my = lax.axis_index("i")

    xb = x[0].astype(jnp.bfloat16)
    Wqb = Wq.astype(jnp.bfloat16)
    Wob = Wo.astype(jnp.bfloat16)
    K_s = lax.dynamic_slice_in_dim(K_ext[0], my * HQ, HQ, 1)
    V_s = lax.dynamic_slice_in_dim(V_ext[0], my * HQ, HQ, 1)
    KT = jnp.transpose(K_s.astype(jnp.bfloat16), (1, 2, 0))
    Vh = jnp.transpose(V_s.astype(jnp.bfloat16), (1, 0, 2))

    def body(x_ref, wq_ref, kt_ref, v_ref, wo_ref, out_ref, bias_ref,
             red_send, red_buf, bc_send, bc_buf,
             red_send_sems, red_recv_sems, bc_send_sems, bc_recv_sems):
        my_pos = lax.axis_index("i")

        barrier = pltpu.get_barrier_semaphore()
        for j in range(1, N_DEV):
            pl.semaphore_signal(
                barrier, inc=1,
                device_id=(lax.rem(my_pos + j, N_DEV),),
                device_id_type=pl.DeviceIdType.MESH,
            )

        row = lax.broadcasted_iota(jnp.int32, (SQ, SKV), 0) // BLK
        col = lax.broadcasted_iota(jnp.int32, (SQ, SKV), 1) // BLK
        keep = (row == col) | (col == 0) | ((row + col) % 3 == 0)
        bias_ref[...] = jnp.where(keep, 0.0, -1e9)

        def compute_chunk(c):
            row0 = c * R
            x_rows = x_ref[pl.ds(row0, R), :]
            bias = bias_ref[pl.ds(row0, R), :]
            q_all = jnp.dot(x_rows, wq_ref[...],
                            preferred_element_type=jnp.float32)
            q_all = (q_all * SCALE).astype(jnp.bfloat16)
            ctxs = []
            for h in range(HQ):
                s = jnp.dot(q_all[:, h * DH:(h + 1) * DH], kt_ref[h],
                            preferred_element_type=jnp.float32)
                s = s + bias
                m = jnp.max(s, axis=1, keepdims=True)
                e = jnp.exp(s - m)
                den = jnp.sum(e, axis=1, keepdims=True)
                w = (e / den).astype(jnp.bfloat16)
                ctxs.append(jnp.dot(w, v_ref[h],
                                    preferred_element_type=jnp.float32)
                            .astype(jnp.bfloat16))
            ctx_all = jnp.concatenate(ctxs, axis=1)
            return jnp.dot(ctx_all, wo_ref[...],
                           preferred_element_type=jnp.float32)

        red_rdmas = []
        for j in range(1, N_DEV):
            c = lax.rem(my_pos + j, N_DEV)
            part = compute_chunk(c)
            red_send[j - 1] = part.astype(jnp.bfloat16)
            if j == 1:
                pl.semaphore_wait(barrier, N_DEV - 1)
            rdma = pltpu.make_async_remote_copy(
                src_ref=red_send.at[j - 1],
                dst_ref=red_buf.at[j - 1],
                send_sem=red_send_sems.at[j - 1],
                recv_sem=red_recv_sems.at[j - 1],
                device_id=(c,),
                device_id_type=pl.DeviceIdType.MESH,
            )
            rdma.start()
            red_rdmas.append(rdma)

        own = compute_chunk(my_pos)
        total = own
        for k in range(N_DEV - 1):
            red_rdmas[k].wait_recv()
            total = total + red_buf[k].astype(jnp.float32)
        out_ref[pl.ds(my_pos * R, R), :] = total

        bc_send[...] = total.astype(jnp.bfloat16)
        bc_rdmas = []
        for j in range(1, N_DEV):
            rdma = pltpu.make_async_remote_copy(
                src_ref=bc_send,
                dst_ref=bc_buf.at[j - 1],
                send_sem=bc_send_sems.at[j - 1],
                recv_sem=bc_recv_sems.at[j - 1],
                device_id=(lax.rem(my_pos + j, N_DEV),),
                device_id_type=pl.DeviceIdType.MESH,
            )
            rdma.start()
            bc_rdmas.append(rdma)
        for k in range(N_DEV - 1):
            owner = lax.rem(my_pos + N_DEV - 1 - k, N_DEV)
            bc_rdmas[k].wait_recv()
            out_ref[pl.ds(owner * R, R), :] = bc_buf[k].astype(jnp.float32)

        for rdma in red_rdmas + bc_rdmas:
            rdma.wait_send()

    out = pl.pallas_call(
        body,
        out_shape=jax.ShapeDtypeStruct((SQ, DMODEL), jnp.float32),
        in_specs=[pl.BlockSpec(memory_space=pltpu.VMEM)] * 5,
        out_specs=pl.BlockSpec(memory_space=pltpu.VMEM),
        scratch_shapes=[
            pltpu.VMEM((SQ, SKV), jnp.float32),
            pltpu.VMEM((N_DEV - 1, R, DMODEL), jnp.bfloat16),
            pltpu.VMEM((N_DEV - 1, R, DMODEL), jnp.bfloat16),
            pltpu.VMEM((R, DMODEL), jnp.bfloat16),
            pltpu.VMEM((N_DEV - 1, R, DMODEL), jnp.bfloat16),
            pltpu.SemaphoreType.DMA((N_DEV - 1,)),
            pltpu.SemaphoreType.DMA((N_DEV - 1,)),
            pltpu.SemaphoreType.DMA((N_DEV - 1,)),
            pltpu.SemaphoreType.DMA((N_DEV - 1,)),
        ],
        compiler_params=pltpu.CompilerParams(collective_id=0),
    )(xb, Wqb, KT, Vh, Wob)
    return out[None]
